# Initial kernel scaffold; baseline (speedup 1.0000x reference)
import jax
import jax.numpy as jnp
from jax.experimental import pallas as pl


def kernel(atoms, state, bonds, bond_atom_1, bond_atom_2, batch_mark_for_atoms, batch_mark_for_bonds, params):
    raise NotImplementedError("write your pallas kernel here")


# speedup vs baseline: 3.2625x; 3.2625x over previous
import functools

import jax
import jax.numpy as jnp
from jax import lax
from jax.experimental import pallas as pl
from jax.experimental.pallas import tpu as pltpu
from jax.experimental.pallas import tpu_sc as plsc

N_NODES = 10000
N_EDGES = 320000
N_GRAPHS = 16

NC, NS = 2, 16
NW = NC * NS
SST = 200
BE = 2000
GRID_E = N_EDGES // BE
BN = 2000
GRID_N = N_NODES // BN

_TC_PARAMS = pltpu.CompilerParams(vmem_limit_bytes=100 * 1024 * 1024)

_SELU_A = 1.6732632423543772
_SELU_S = 1.0507009873554805


def _selu(x):
    return _SELU_S * jnp.where(x > 0, x, _SELU_A * (jnp.exp(x) - 1.0))


def _sigmoid(x):
    return 1.0 / (1.0 + jnp.exp(-x))


def _dot(a, b):
    return jax.lax.dot_general(a, b, (((1,), (0,)), ((), ())),
                               preferred_element_type=jnp.float32)


def _dot_t(a, b):
    return jax.lax.dot_general(a, b, (((0,), (0,)), ((), ())),
                               preferred_element_type=jnp.float32)


def _sc_gather(table, idx3d):
    ngrp = idx3d.shape[0]
    B = ngrp * 256
    nit = (ngrp + NW - 1) // NW
    mesh = plsc.VectorSubcoreMesh(core_axis_name="c", subcore_axis_name="s")

    @functools.partial(
        pl.kernel, mesh=mesh,
        out_type=jax.ShapeDtypeStruct((B, 128), jnp.float32),
        scratch_types=[
            [pltpu.VMEM((128,), jnp.int32)] * 2,
            pltpu.VMEM((256, 128), jnp.float32),
            pltpu.VMEM_SHARED((N_NODES, 128), jnp.float32),
            pltpu.SemaphoreType.DMA,
        ],
    )
    def k(table_hbm, idx_hbm, out_hbm, idx_v, rows_v, shared, sem):
        c = lax.axis_index("c")
        s = lax.axis_index("s")
        wid = s * NC + c

        @pl.when(s < 10)
        def _():
            def stg(j, carry):
                off = s * 1000 + j * SST
                pltpu.sync_copy(table_hbm.at[pl.ds(off, SST)],
                                rows_v.at[pl.ds(0, SST)])
                pltpu.sync_copy(rows_v.at[pl.ds(0, SST)],
                                shared.at[pl.ds(off, SST)])
                return carry

            lax.fori_loop(0, 1000 // SST, stg, 0)

        plsc.subcore_barrier()

        def body(it, carry):
            cid = wid + it * NW

            @pl.when(cid < ngrp)
            def _():
                for j in range(2):
                    pltpu.sync_copy(idx_hbm.at[cid, j], idx_v[j])
                for j in range(2):
                    pltpu.async_copy(shared.at[idx_v[j]],
                                     rows_v.at[pl.ds(j * 128, 128)],
                                     sem).wait()
                pltpu.sync_copy(rows_v, out_hbm.at[pl.ds(cid * 256, 256)])

            return carry

        lax.fori_loop(0, nit, body, 0)

    return k(table, idx3d)


def _sc_scatter_add(vals, idx3d, zeros_chunk):
    ngrp = idx3d.shape[0]
    nit = (ngrp + NW - 1) // NW
    mesh = plsc.VectorSubcoreMesh(core_axis_name="c", subcore_axis_name="s")

    @functools.partial(
        pl.kernel, mesh=mesh,
        out_type=jax.ShapeDtypeStruct((2 * N_NODES, 128), jnp.float32),
        scratch_types=[
            [pltpu.VMEM((128,), jnp.int32)] * 2,
            pltpu.VMEM((256, 128), jnp.float32),
            pltpu.VMEM_SHARED((N_NODES, 128), jnp.float32),
        ],
    )
    def k(vals_hbm, idx_hbm, zeros_hbm, out_hbm, idx_v, vals_v, shared):
        c = lax.axis_index("c")
        s = lax.axis_index("s")
        wid = s * NC + c

        @pl.when(s < 10)
        def _():
            pltpu.sync_copy(zeros_hbm, vals_v.at[pl.ds(0, SST)])

            def zr(j, carry):
                pltpu.sync_copy(vals_v.at[pl.ds(0, SST)],
                                shared.at[pl.ds(s * 1000 + j * SST, SST)])
                return carry

            lax.fori_loop(0, 1000 // SST, zr, 0)

        plsc.subcore_barrier()

        def body(it, carry):
            cid = wid + it * NW

            @pl.when(cid < ngrp)
            def _():
                for j in range(2):
                    pltpu.sync_copy(idx_hbm.at[cid, j], idx_v[j])
                pltpu.sync_copy(vals_hbm.at[pl.ds(cid * 256, 256)], vals_v)
                for j in range(2):
                    pltpu.sync_copy(vals_v.at[pl.ds(j * 128, 128)],
                                    shared.at[idx_v[j]], add=True)

            return carry

        lax.fori_loop(0, nit, body, 0)
        plsc.subcore_barrier()

        @pl.when(s < 10)
        def _():
            def wr(j, carry):
                off = s * 1000 + j * SST
                pltpu.sync_copy(shared.at[pl.ds(off, SST)],
                                vals_v.at[pl.ds(0, SST)])
                pltpu.sync_copy(vals_v.at[pl.ds(0, SST)],
                                out_hbm.at[pl.ds(c * N_NODES + off, SST)])
                return carry

            lax.fori_loop(0, 1000 // SST, wr, 0)

    return k(vals, idx3d, zeros_chunk)


def _sc_count(idx3d, ones_chunk, zeros_chunk):
    ngrp = idx3d.shape[0]
    nit = (ngrp + NW - 1) // NW
    mesh = plsc.VectorSubcoreMesh(core_axis_name="c", subcore_axis_name="s")

    @functools.partial(
        pl.kernel, mesh=mesh,
        out_type=jax.ShapeDtypeStruct((2 * N_NODES, 128), jnp.float32),
        scratch_types=[
            [pltpu.VMEM((128,), jnp.int32)] * 2,
            pltpu.VMEM((256, 128), jnp.float32),
            pltpu.VMEM_SHARED((N_NODES, 128), jnp.float32),
        ],
    )
    def k(idx_hbm, ones_hbm, zeros_hbm, out_hbm, idx_v, buf_v, shared):
        c = lax.axis_index("c")
        s = lax.axis_index("s")
        wid = s * NC + c

        @pl.when(s < 10)
        def _():
            pltpu.sync_copy(zeros_hbm, buf_v.at[pl.ds(0, SST)])

            def zr(j, carry):
                pltpu.sync_copy(buf_v.at[pl.ds(0, SST)],
                                shared.at[pl.ds(s * 1000 + j * SST, SST)])
                return carry

            lax.fori_loop(0, 1000 // SST, zr, 0)

        plsc.subcore_barrier()

        pltpu.sync_copy(ones_hbm, buf_v.at[pl.ds(0, 128)])

        def body(it, carry):
            cid = wid + it * NW

            @pl.when(cid < ngrp)
            def _():
                for j in range(2):
                    pltpu.sync_copy(idx_hbm.at[cid, j], idx_v[j])
                for j in range(2):
                    pltpu.sync_copy(buf_v.at[pl.ds(0, 128)],
                                    shared.at[idx_v[j]], add=True)

            return carry

        lax.fori_loop(0, nit, body, 0)
        plsc.subcore_barrier()

        @pl.when(s < 10)
        def _():
            def wr(j, carry):
                off = s * 1000 + j * SST
                pltpu.sync_copy(shared.at[pl.ds(off, SST)],
                                buf_v.at[pl.ds(0, SST)])
                pltpu.sync_copy(buf_v.at[pl.ds(0, SST)],
                                out_hbm.at[pl.ds(c * N_NODES + off, SST)])
                return carry

            lax.fori_loop(0, 1000 // SST, wr, 0)

    return k(idx3d, ones_chunk, zeros_chunk)


def _full(shape):
    return pl.BlockSpec(shape, lambda *_: tuple(0 for _ in shape))


def _pre_kernel(atoms_idx2d, state, emb, aw1, ab1, aw2, ab2, sw1, sb1, sw2, sb2):
    def body(idx_ref, st_ref, emb_ref, aw1_r, ab1_r, aw2_r, ab2_r,
             sw1_r, sb1_r, sw2_r, sb2_r, atoms_out, atoms32_out, state_out):
        idx = idx_ref[...]
        oh = (idx == lax.broadcasted_iota(jnp.int32, (N_NODES, 95), 1)
              ).astype(jnp.float32)
        e = _dot(oh, emb_ref[...])
        h = _selu(_dot(e, aw1_r[...]) + ab1_r[...])
        a32 = _selu(_dot(h, aw2_r[...]) + ab2_r[...])
        atoms_out[...] = jnp.concatenate(
            [a32, jnp.zeros((N_NODES, 96), jnp.float32)], axis=1)
        atoms32_out[...] = a32
        hs = _selu(_dot(st_ref[...], sw1_r[...]) + sb1_r[...])
        state_out[...] = _selu(_dot(hs, sw2_r[...]) + sb2_r[...])

    return pl.pallas_call(
        body,
        grid=(1,),
        in_specs=[_full((N_NODES, 1)), _full((1, 2)), _full((95, 16)),
                  _full((16, 64)), _full((1, 64)), _full((64, 32)), _full((1, 32)),
                  _full((2, 64)), _full((1, 64)), _full((64, 32)), _full((1, 32))],
        out_specs=[_full((N_NODES, 128)), _full((N_NODES, 32)),
                   _full((1, 32))],
        out_shape=[jax.ShapeDtypeStruct((N_NODES, 128), jnp.float32),
                   jax.ShapeDtypeStruct((N_NODES, 32), jnp.float32),
                   jax.ShapeDtypeStruct((1, 32), jnp.float32)],
        compiler_params=_TC_PARAMS,
    )(atoms_idx2d, state, emb, aw1, ab1, aw2, ab2, sw1, sb1, sw2, sb2)


def _bondpre_kernel(bonds, w1, b1, w2, b2):
    def body(x_ref, w1_r, b1_r, w2_r, b2_r, out_ref):
        h = _selu(_dot(x_ref[...], w1_r[...]) + b1_r[...])
        out_ref[...] = _selu(_dot(h, w2_r[...]) + b2_r[...])

    return pl.pallas_call(
        body,
        grid=(GRID_E,),
        in_specs=[pl.BlockSpec((BE, 100), lambda i: (i, 0)),
                  _full((100, 64)), _full((1, 64)), _full((64, 32)), _full((1, 32))],
        out_specs=pl.BlockSpec((BE, 32), lambda i: (i, 0)),
        out_shape=jax.ShapeDtypeStruct((N_EDGES, 32), jnp.float32),
        compiler_params=_TC_PARAMS,
    )(bonds, w1, b1, w2, b2)


def _phi_e_kernel(ga, rb, bonds_old, sv, pe, ff_next):
    (we1, be1), (we2, be2), (we3, be3) = pe
    with_ff = ff_next is not None

    def body(a1_r, a2_r, rb_r, bo_r, sv_r, we1_r, be1_r, we2_r, be2_r,
             we3_r, be3_r, *rest):
        if with_ff:
            wf1_r, bf1_r, wf2_r, bf2_r = rest[:4]
            bn_o, nb_o, rbn_o, bsum_o = rest[4:]
        else:
            bn_o, nb_o, bsum_o = rest

        w1 = we1_r[...]
        x = (_dot(a1_r[:, 0:32], w1[0:32]) + _dot(a2_r[:, 0:32], w1[32:64])
             + _dot(rb_r[...], w1[64:96]) + _dot(sv_r[...], w1[96:128])
             + be1_r[...])
        h1 = _selu(x)
        h2 = _selu(_dot(h1, we2_r[...]) + be2_r[...])
        bn = _selu(_dot(h2, we3_r[...]) + be3_r[...])
        bn_o[...] = jnp.concatenate(
            [bn, jnp.zeros((BE, 96), jnp.float32)], axis=1)
        nb = bo_r[...] + bn
        nb_o[...] = nb
        if with_ff:
            hf = _selu(_dot(nb, wf1_r[...]) + bf1_r[...])
            rbn_o[...] = _selu(_dot(hf, wf2_r[...]) + bf2_r[...])

        @pl.when(pl.program_id(0) == 0)
        def _():
            bsum_o[...] = jnp.zeros_like(bsum_o)

        bsum_o[...] += jnp.sum(bn, axis=0, keepdims=True)

    eb = lambda i: (i, 0)
    eb2 = lambda i: (i + GRID_E, 0)
    in_specs = [pl.BlockSpec((BE, 128), eb), pl.BlockSpec((BE, 128), eb2)] + [
        pl.BlockSpec((BE, 32), eb)] * 2 + [
        _full((1, 32)), _full((128, 64)), _full((1, 64)),
        _full((64, 64)), _full((1, 64)), _full((64, 32)), _full((1, 32))]
    args = [ga, ga, rb, bonds_old, sv, we1, be1, we2, be2, we3, be3]
    out_specs = [pl.BlockSpec((BE, 128), eb), pl.BlockSpec((BE, 32), eb)]
    out_shape = [jax.ShapeDtypeStruct((N_EDGES, 128), jnp.float32),
                 jax.ShapeDtypeStruct((N_EDGES, 32), jnp.float32)]
    if with_ff:
        (wf1, bf1), (wf2, bf2) = ff_next
        in_specs += [_full((32, 64)), _full((1, 64)), _full((64, 32)), _full((1, 32))]
        args += [wf1, bf1, wf2, bf2]
        out_specs.append(pl.BlockSpec((BE, 32), eb))
        out_shape.append(jax.ShapeDtypeStruct((N_EDGES, 32), jnp.float32))
    out_specs.append(pl.BlockSpec((1, 32), lambda i: (0, 0)))
    out_shape.append(jax.ShapeDtypeStruct((1, 32), jnp.float32))

    res = pl.pallas_call(
        body, grid=(GRID_E,), in_specs=in_specs, out_specs=out_specs,
        out_shape=out_shape, compiler_params=_TC_PARAMS,
    )(*args)
    if with_ff:
        bn, nb, rbn, bsum = res
    else:
        bn, nb, bsum = res
        rbn = None
    return bn, nb, rbn, bsum


def _node_kernel(bta2, cnt2, ra, atoms_old, sv, pv, ff_a):
    (wv1, bv1), (wv2, bv2), (wv3, bv3) = pv
    with_ff = ff_a is not None

    def body(bta0_r, bta1_r, cnt0_r, cnt1_r, ra_r, ao_r, sv_r,
             wv1_r, bv1_r, wv2_r, bv2_r, wv3_r, bv3_r, *rest):
        if with_ff:
            wa1_r, ba1_r, wa2_r, ba2_r = rest[:4]
            na_o, ran_o, asum_o = rest[4:]
        else:
            na_o, asum_o = rest

        bta = bta0_r[:, 0:32] + bta1_r[:, 0:32]
        cnt = cnt0_r[:, 0:1] + cnt1_r[:, 0:1]
        bta = bta / cnt
        w1 = wv1_r[...]
        x = (_dot(bta, w1[0:32]) + _dot(ra_r[:, 0:32], w1[32:64])
             + _dot(sv_r[...], w1[64:96]) + bv1_r[...])
        h = _selu(x)
        h = _selu(_dot(h, wv2_r[...]) + bv2_r[...])
        an = _selu(_dot(h, wv3_r[...]) + bv3_r[...])
        na = ao_r[...] + an
        na_o[...] = na
        if with_ff:
            ha = _selu(_dot(na, wa1_r[...]) + ba1_r[...])
            ran_o[...] = jnp.concatenate(
                [_selu(_dot(ha, wa2_r[...]) + ba2_r[...]),
                 jnp.zeros((BN, 96), jnp.float32)], axis=1)

        @pl.when(pl.program_id(0) == 0)
        def _():
            asum_o[...] = jnp.zeros_like(asum_o)

        asum_o[...] += jnp.sum(an, axis=0, keepdims=True)

    nb = lambda i: (i, 0)
    nb1 = lambda i: (i + GRID_N, 0)
    in_specs = [pl.BlockSpec((BN, 128), nb), pl.BlockSpec((BN, 128), nb1),
                pl.BlockSpec((BN, 128), nb), pl.BlockSpec((BN, 128), nb1),
                pl.BlockSpec((BN, 128), nb), pl.BlockSpec((BN, 32), nb),
                _full((1, 32)),
                _full((96, 64)), _full((1, 64)), _full((64, 64)), _full((1, 64)),
                _full((64, 32)), _full((1, 32))]
    args = [bta2, bta2, cnt2, cnt2, ra, atoms_old, sv,
            wv1, bv1, wv2, bv2, wv3, bv3]
    out_specs = [pl.BlockSpec((BN, 32), nb)]
    out_shape = [jax.ShapeDtypeStruct((N_NODES, 32), jnp.float32)]
    if with_ff:
        (wa1, ba1_), (wa2, ba2_) = ff_a
        in_specs += [_full((32, 64)), _full((1, 64)), _full((64, 32)),
                     _full((1, 32))]
        args += [wa1, ba1_, wa2, ba2_]
        out_specs.append(pl.BlockSpec((BN, 128), nb))
        out_shape.append(jax.ShapeDtypeStruct((N_NODES, 128), jnp.float32))
    out_specs.append(pl.BlockSpec((1, 32), lambda i: (0, 0)))
    out_shape.append(jax.ShapeDtypeStruct((1, 32), jnp.float32))

    res = pl.pallas_call(
        body, grid=(GRID_N,), in_specs=in_specs, out_specs=out_specs,
        out_shape=out_shape, compiler_params=_TC_PARAMS,
    )(*args)
    if with_ff:
        return res
    return res[0], None, res[1]


def _state_kernel(bsum, asum, sv, state_old, pu, ff_s):
    (wu1, bu1), (wu2, bu2), (wu3, bu3) = pu
    with_ff = ff_s is not None

    def body(bsum_r, asum_r, sv_r, so_r,
             wu1_r, bu1_r, wu2_r, bu2_r, wu3_r, bu3_r, *rest):
        if with_ff:
            ws1_r, bs1_r, ws2_r, bs2_r = rest[:4]
            ns_o, rsn_o = rest[4:]
        else:
            (ns_o,) = rest
        bmean = bsum_r[...] * (1.0 / N_EDGES)
        amean = asum_r[...] * (1.0 / N_NODES)
        wu = wu1_r[...]
        xu = (_dot(bmean, wu[0:32]) + _dot(amean, wu[32:64])
              + _dot(sv_r[...], wu[64:96]) + bu1_r[...])
        hu = _selu(xu)
        hu = _selu(_dot(hu, wu2_r[...]) + bu2_r[...])
        un = _selu(_dot(hu, wu3_r[...]) + bu3_r[...])
        ns = so_r[...] + un
        ns_o[...] = ns
        if with_ff:
            hs = _selu(_dot(ns, ws1_r[...]) + bs1_r[...])
            rsn_o[...] = _selu(_dot(hs, ws2_r[...]) + bs2_r[...])

    in_specs = [_full((1, 32))] * 4 + [
        _full((96, 64)), _full((1, 64)), _full((64, 64)), _full((1, 64)),
        _full((64, 32)), _full((1, 32))]
    args = [bsum, asum, sv, state_old, wu1, bu1, wu2, bu2, wu3, bu3]
    out_specs = [_full((1, 32))]
    out_shape = [jax.ShapeDtypeStruct((1, 32), jnp.float32)]
    if with_ff:
        (ws1, bs1), (ws2, bs2) = ff_s
        in_specs += [_full((32, 64)), _full((1, 64)), _full((64, 32)),
                     _full((1, 32))]
        args += [ws1, bs1, ws2, bs2]
        out_specs.append(_full((1, 32)))
        out_shape.append(jax.ShapeDtypeStruct((1, 32), jnp.float32))

    res = pl.pallas_call(
        body, grid=(1,), in_specs=in_specs, out_specs=out_specs,
        out_shape=out_shape, compiler_params=_TC_PARAMS,
    )(*args)
    if with_ff:
        return res
    return res[0], None


def _lstm_step_kernel(h, c, q, R, s, wih, whh, b, first):
    def body(h_r, c_r, q_r, R_r, s_r, wih_r, whh_r, b_r, ho, co, qo):
        if first:
            qs = jnp.zeros((N_GRAPHS, 64), jnp.float32)
        else:
            recip = 1.0 / s_r[...]
            eye = (lax.broadcasted_iota(jnp.int32, (16, 16), 0)
                   == lax.broadcasted_iota(jnp.int32, (16, 16), 1)
                   ).astype(jnp.float32)
            d = eye * recip
            r = _dot(d, R_r[...])
            qs = jnp.concatenate([q_r[...], r], axis=1)
        gates = _dot(qs, wih_r[...]) + _dot(h_r[...], whh_r[...]) + b_r[...]
        gi = gates[:, 0:32]
        gf = gates[:, 32:64]
        gg = gates[:, 64:96]
        go = gates[:, 96:128]
        cn = _sigmoid(gf) * c_r[...] + _sigmoid(gi) * jnp.tanh(gg)
        hn = _sigmoid(go) * jnp.tanh(cn)
        co[...] = cn
        ho[...] = hn
        qo[...] = hn

    return pl.pallas_call(
        body, grid=(1,),
        in_specs=[_full((16, 32)), _full((16, 32)), _full((16, 32)),
                  _full((16, 32)), _full((1, 16)),
                  _full((64, 128)), _full((32, 128)), _full((1, 128))],
        out_specs=[_full((16, 32))] * 3,
        out_shape=[jax.ShapeDtypeStruct((16, 32), jnp.float32)] * 3,
        compiler_params=_TC_PARAMS,
    )(h, c, q, R, s, wih, whh, b)


def _s2s_max_kernel(x, starts, ends, q):
    def body(x_r, st_r, en_r, q_r, m_o):
        i = pl.program_id(0)
        rows = lax.broadcasted_iota(jnp.int32, (BE, 16), 0) + i * BE
        mask = (rows >= st_r[...]) & (rows < en_r[...])
        oh = mask.astype(jnp.float32)
        qr = _dot(oh, q_r[...])
        e = jnp.sum(x_r[...] * qr, axis=1, keepdims=True)
        em = jnp.where(mask, e, -jnp.inf)
        bm = jnp.max(em, axis=0, keepdims=True)

        @pl.when(i == 0)
        def _():
            m_o[...] = jnp.full((1, 16), -jnp.inf, jnp.float32)

        m_o[...] = jnp.maximum(m_o[...], bm)

    return pl.pallas_call(
        body, grid=(GRID_E,),
        in_specs=[pl.BlockSpec((BE, 32), lambda i: (i, 0)),
                  _full((1, 16)), _full((1, 16)), _full((16, 32))],
        out_specs=pl.BlockSpec((1, 16), lambda i: (0, 0)),
        out_shape=jax.ShapeDtypeStruct((1, 16), jnp.float32),
        compiler_params=_TC_PARAMS,
    )(x, starts, ends, q)


def _s2s_sum_kernel(x, starts, ends, q, m):
    def body(x_r, st_r, en_r, q_r, m_r, s_o, R_o):
        i = pl.program_id(0)
        rows = lax.broadcasted_iota(jnp.int32, (BE, 16), 0) + i * BE
        mask = (rows >= st_r[...]) & (rows < en_r[...])
        oh = mask.astype(jnp.float32)
        xv = x_r[...]
        qr = _dot(oh, q_r[...])
        e = jnp.sum(xv * qr, axis=1, keepdims=True)
        a16 = jnp.where(mask, jnp.exp(e - m_r[...]), 0.0)
        sb = jnp.sum(a16, axis=0, keepdims=True)
        arow = jnp.sum(a16, axis=1, keepdims=True)
        Rb = _dot_t(a16, xv * 1.0)
        del arow

        @pl.when(i == 0)
        def _():
            s_o[...] = jnp.zeros_like(s_o)
            R_o[...] = jnp.zeros_like(R_o)

        s_o[...] += sb
        R_o[...] += Rb

    return pl.pallas_call(
        body, grid=(GRID_E,),
        in_specs=[pl.BlockSpec((BE, 32), lambda i: (i, 0)),
                  _full((1, 16)), _full((1, 16)), _full((16, 32)), _full((1, 16))],
        out_specs=[pl.BlockSpec((1, 16), lambda i: (0, 0)),
                   pl.BlockSpec((16, 32), lambda i: (0, 0))],
        out_shape=[jax.ShapeDtypeStruct((1, 16), jnp.float32),
                   jax.ShapeDtypeStruct((16, 32), jnp.float32)],
        compiler_params=_TC_PARAMS,
    )(x, starts, ends, q, m)


def _final_kernel(x, starts, ends, wih, whh, b, bq, bR, bs, state_fin,
                  ow1, ob1, ow2, ob2, ow3, ob3):
    def body(x_r, st_r, en_r, wih_r, whh_r, b_r, bq_r, bR_r, bs_r, sf_r,
             ow1_r, ob1_r, ow2_r, ob2_r, ow3_r, ob3_r, out_o):
        xv = x_r[...]
        rows = lax.broadcasted_iota(jnp.int32, (N_NODES, 16), 0)
        mask = (rows >= st_r[...]) & (rows < en_r[...])
        oh = mask.astype(jnp.float32)
        eye = (lax.broadcasted_iota(jnp.int32, (16, 16), 0)
               == lax.broadcasted_iota(jnp.int32, (16, 16), 1)
               ).astype(jnp.float32)

        h = jnp.zeros((16, 32), jnp.float32)
        c = jnp.zeros((16, 32), jnp.float32)
        qs = jnp.zeros((16, 64), jnp.float32)
        for _step in range(3):
            gates = _dot(qs, wih_r[...]) + _dot(h, whh_r[...]) + b_r[...]
            gi = gates[:, 0:32]
            gf = gates[:, 32:64]
            gg = gates[:, 64:96]
            go = gates[:, 96:128]
            c = _sigmoid(gf) * c + _sigmoid(gi) * jnp.tanh(gg)
            h = _sigmoid(go) * jnp.tanh(c)
            q = h
            qr = _dot(oh, q)
            e = jnp.sum(xv * qr, axis=1, keepdims=True)
            em = jnp.where(mask, e, -jnp.inf)
            m = jnp.max(em, axis=0, keepdims=True)
            a16 = jnp.where(mask, jnp.exp(e - m), 0.0)
            s = jnp.sum(a16, axis=0, keepdims=True)
            R = _dot_t(a16, xv)
            r = _dot(eye * (1.0 / s), R)
            qs = jnp.concatenate([q, r], axis=1)

        br = _dot(eye * (1.0 / bs_r[...]), bR_r[...])
        bqs = jnp.concatenate([bq_r[...], br], axis=1)
        sfb = jnp.broadcast_to(sf_r[...], (16, 32))
        g = jnp.concatenate([bqs, qs, sfb], axis=1)
        o1 = _selu(_dot(g, ow1_r[...]) + ob1_r[...])
        o2 = _selu(_dot(o1, ow2_r[...]) + ob2_r[...])
        out_o[...] = _selu(_dot(o2, ow3_r[...]) + ob3_r[...])

    return pl.pallas_call(
        body, grid=(1,),
        in_specs=[_full((N_NODES, 32)), _full((1, 16)), _full((1, 16)),
                  _full((64, 128)), _full((32, 128)), _full((1, 128)),
                  _full((16, 32)), _full((16, 32)), _full((1, 16)), _full((1, 32)),
                  _full((160, 128)), _full((1, 128)),
                  _full((128, 64)), _full((1, 64)),
                  _full((64, 200)), _full((1, 200))],
        out_specs=_full((16, 200)),
        out_shape=jax.ShapeDtypeStruct((16, 200), jnp.float32),
        compiler_params=_TC_PARAMS,
    )(x, starts, ends, wih, whh, b, bq, bR, bs, state_fin,
      ow1, ob1, ow2, ob2, ow3, ob3)


def _row(b):
    return b.reshape(1, -1)


def _ff2(ps):
    (w1, b1), (w2, b2) = ps
    return (w1, _row(b1)), (w2, _row(b2))


def _ff3(ps):
    (w1, b1), (w2, b2), (w3, b3) = ps
    return (w1, _row(b1)), (w2, _row(b2)), (w3, _row(b3))


def _seg_bounds(marks, n):
    g = jnp.arange(N_GRAPHS, dtype=marks.dtype)
    starts = jnp.searchsorted(marks, g, side="left").astype(jnp.int32)
    ends = jnp.searchsorted(marks, g, side="right").astype(jnp.int32)
    return starts.reshape(1, 16), ends.reshape(1, 16)


def kernel(atoms, state, bonds, bond_atom_1, bond_atom_2,
           batch_mark_for_atoms, batch_mark_for_bonds, params):
    ba1 = bond_atom_1.astype(jnp.int32)
    ba2 = bond_atom_2.astype(jnp.int32)
    idx2 = jnp.concatenate([ba1, ba2], axis=0).reshape(-1, 2, 128)
    ba1_3d = ba1.reshape(-1, 2, 128)
    zeros128 = jnp.zeros((SST, 128), jnp.float32)
    ones128 = jnp.ones((128, 128), jnp.float32)

    p = params
    (aw1, ab1), (aw2, ab2) = _ff2(p["atom_preblock"])
    (sw1, sb1), (sw2, sb2) = _ff2(p["state_preblock"])
    (bw1, bb1), (bw2, bb2) = _ff2(p["bond_preblock"])

    atoms_pre, atoms_pre32, state_pre = _pre_kernel(
        atoms.astype(jnp.int32).reshape(N_NODES, 1), state, p["embedding"],
        aw1, ab1, aw2, ab2, sw1, sb1, sw2, sb2)
    bonds_pre = _bondpre_kernel(bonds, bw1, bb1, bw2, bb2)

    cnt2 = _sc_count(ba1_3d, ones128, zeros128)

    bonds_c, atoms_c, state_c = bonds_pre, atoms_pre32, state_pre
    rb, ra, sv = bonds_pre, atoms_pre, state_pre

    blocks = p["blocks"]
    for bi in range(4):
        blk = blocks[bi]
        pe = _ff3(blk["phi_e"])
        pv = _ff3(blk["phi_v"])
        pu = _ff3(blk["phi_u"])
        if bi + 1 < 4:
            nxt = blocks[bi + 1]
            ff_b = _ff2(nxt["bonds_ff"])
            ff_a = _ff2(nxt["atoms_ff"])
            ff_s = _ff2(nxt["state_ff"])
        else:
            ff_b = ff_a = ff_s = None

        ga = _sc_gather(ra, idx2)

        bn, bonds_c, rb_next, bsum = _phi_e_kernel(
            ga, rb, bonds_c, sv, pe, ff_b)

        bta2 = _sc_scatter_add(bn, ba1_3d, zeros128)

        atoms_c, ra_next, asum = _node_kernel(
            bta2, cnt2, ra, atoms_c, sv, pv, ff_a)
        state_c, sv_next = _state_kernel(bsum, asum, sv, state_c, pu, ff_s)

        rb, ra, sv = rb_next, ra_next, sv_next

    e_starts, e_ends = _seg_bounds(batch_mark_for_bonds, N_EDGES)
    a_starts, a_ends = _seg_bounds(batch_mark_for_atoms, N_NODES)

    lse = p["set2set_e"]
    wih_e, whh_e = lse["W_ih"], lse["W_hh"]
    b_e = _row(lse["b_ih"] + lse["b_hh"])
    lsv = p["set2set_v"]
    wih_v, whh_v = lsv["W_ih"], lsv["W_hh"]
    b_v = _row(lsv["b_ih"] + lsv["b_hh"])

    h = jnp.zeros((16, 32), jnp.float32)
    c = jnp.zeros((16, 32), jnp.float32)
    q = jnp.zeros((16, 32), jnp.float32)
    R = jnp.zeros((16, 32), jnp.float32)
    s = jnp.ones((1, 16), jnp.float32)
    for step in range(3):
        h, c, q = _lstm_step_kernel(h, c, q, R, s, wih_e, whh_e, b_e,
                                    first=(step == 0))
        m = _s2s_max_kernel(bonds_c, e_starts, e_ends, q)
        s, R = _s2s_sum_kernel(bonds_c, e_starts, e_ends, q, m)

    (ow1, ob1), (ow2, ob2), (ow3, ob3) = _ff3(p["output"])
    out = _final_kernel(atoms_c, a_starts, a_ends, wih_v, whh_v, b_v,
                        q, R, s, state_c, ow1, ob1, ow2, ob2, ow3, ob3)
    return out
